# baseline (speedup 1.0000x reference)
import functools

import jax
import jax.numpy as jnp
from jax import lax
from jax.experimental import pallas as pl
from jax.experimental.pallas import tpu as pltpu
from jax.experimental.pallas import tpu_sc as plsc

SEQ = 77
DIM = 512
N_CTX = 4
X_POS = 5

ROWS_PAD = 80
CHUNK = 16
N_CHUNKS = ROWS_PAD // CHUNK
N_WORKERS = 4 * N_CHUNKS

SPW = 32


def _sc_gather_prompts(idx_flat, table):
    info = plsc.get_sparse_core_info()
    nc = info.num_cores
    mesh = plsc.VectorSubcoreMesh(core_axis_name="c", subcore_axis_name="s")

    @functools.partial(
        pl.kernel,
        mesh=mesh,
        out_type=jax.ShapeDtypeStruct((4 * ROWS_PAD, DIM), jnp.float32),
        scratch_types=[
            pltpu.VMEM((CHUNK,), jnp.int32),
            pltpu.VMEM((CHUNK, DIM), jnp.float32),
            pltpu.SemaphoreType.DMA,
        ],
    )
    def k(idx_hbm, table_hbm, out_hbm, idx_v, rows_v, sem):
        wid = lax.axis_index("s") * nc + lax.axis_index("c")

        @pl.when(wid < N_WORKERS)
        def _():
            base = pl.multiple_of(wid * CHUNK, CHUNK)
            pltpu.sync_copy(idx_hbm.at[pl.ds(base, CHUNK)], idx_v)
            pltpu.async_copy(table_hbm.at[idx_v], rows_v, sem).wait()
            pltpu.sync_copy(rows_v, out_hbm.at[pl.ds(base, CHUNK)])

    return k(idx_flat, table)


def _tc_merge(prompts, ctx77, view, time):
    B = view.shape[0]

    def body(prompts_ref, ctx_ref, view_ref, time_ref, out_ref, c8_ref):
        row = lax.broadcasted_iota(jnp.int32, (SEQ, DIM), 0)
        is_ctx = (row >= X_POS) & (row < X_POS + N_CTX)
        ctx_rows = ctx_ref[...]
        for c in range(4):
            out_ref[c] = jnp.where(is_ctx, ctx_rows, prompts_ref[c][0:SEQ, :])
        combo = view_ref[...] * 2 + time_ref[...]
        c8_ref[...] = jnp.broadcast_to(combo[:, None], (B, 8))

    return pl.pallas_call(
        body,
        in_specs=[
            pl.BlockSpec((4, ROWS_PAD, DIM), lambda: (0, 0, 0)),
            pl.BlockSpec((SEQ, DIM), lambda: (0, 0)),
            pl.BlockSpec((B,), lambda: (0,)),
            pl.BlockSpec((B,), lambda: (0,)),
        ],
        out_specs=[
            pl.BlockSpec((4, SEQ, DIM), lambda: (0, 0, 0)),
            pl.BlockSpec((B, 8), lambda: (0, 0)),
        ],
        out_shape=[
            jax.ShapeDtypeStruct((4, SEQ, DIM), jnp.float32),
            jax.ShapeDtypeStruct((B, 8), jnp.int32),
        ],
    )(prompts, ctx77, view, time)


def _sc_broadcast(c8, merged):
    B = c8.shape[0] // 8
    info = plsc.get_sparse_core_info()
    nc = info.num_cores
    mesh = plsc.VectorSubcoreMesh(core_axis_name="c", subcore_axis_name="s")

    @functools.partial(
        pl.kernel,
        mesh=mesh,
        out_type=jax.ShapeDtypeStruct((B, SEQ * DIM), jnp.float32),
        scratch_types=[
            pltpu.VMEM((1, SEQ * DIM), jnp.float32),
            pltpu.VMEM((1, SEQ * DIM), jnp.float32),
            pltpu.VMEM((8 * SPW,), jnp.int32),
            pltpu.SemaphoreType.DMA,
            pltpu.SemaphoreType.DMA,
            pltpu.SemaphoreType.DMA,
        ],
    )
    def k(c8_hbm, merged_hbm, out_hbm, buf0, buf1, c8_v, gsem, wsem0, wsem1):
        bufs = (buf0, buf1)
        wid = lax.axis_index("s") * nc + lax.axis_index("c")
        base = pl.multiple_of(wid * SPW, SPW)
        pltpu.sync_copy(c8_hbm.at[pl.ds(base * 8, 8 * SPW)], c8_v)

        def gather(i, slot):
            idx = c8_v.at[pl.ds(8 * i, 1)]
            return pltpu.async_copy(merged_hbm.at[idx], bufs[slot], gsem)

        def write(i, slot):
            wsem = wsem0 if slot == 0 else wsem1
            return pltpu.async_copy(bufs[slot], out_hbm.at[pl.ds(base + i, 1)], wsem)

        pending = [None, None]
        for i in range(SPW):
            slot = i % 2
            if pending[slot] is not None:
                pending[slot].wait()
            gather(i, slot).wait()
            pending[slot] = write(i, slot)
        pending[0].wait()
        pending[1].wait()

    return k(c8, merged)


def kernel(label, view_label, time_label, tokenized_table, token_embedding, ctx):
    del label
    idx = jnp.pad(tokenized_table.astype(jnp.int32), ((0, 0), (0, ROWS_PAD - SEQ)))
    prompts = _sc_gather_prompts(idx.reshape(-1), token_embedding)
    ctx77 = jnp.pad(ctx, ((X_POS, SEQ - X_POS - N_CTX), (0, 0)))
    merged, c8 = _tc_merge(
        prompts.reshape(4, ROWS_PAD, DIM),
        ctx77,
        view_label.astype(jnp.int32),
        time_label.astype(jnp.int32),
    )
    out = _sc_broadcast(c8.reshape(-1), merged.reshape(4, SEQ * DIM))
    return out.reshape(-1, SEQ, DIM)
